# baseline (speedup 1.0000x reference)
import functools

import jax
import jax.numpy as jnp
from jax import lax
from jax.experimental import pallas as pl
from jax.experimental.pallas import tpu as pltpu
from jax.experimental.pallas import tpu_sc as plsc

_NC = 2
_NS = 16
_NW = _NC * _NS
_CH = 128
_L = 16


_FXS = 4096.0


def _fx16(v):
    return jnp.clip(jnp.round(v * _FXS), -32768.0, 32767.0).astype(jnp.int32)


def _tables_body(x_ref, wd_ref, ws_ref, td_ref, ts_ref):
    xb = x_ref[...]
    d = xb.shape[1]
    td = jnp.dot(xb, wd_ref[...], preferred_element_type=jnp.float32)
    ts = jnp.dot(xb, ws_ref[...], preferred_element_type=jnp.float32)
    td_ref[...] = (_fx16(td[:, :d]) & 0xFFFF) | (_fx16(td[:, d:]) << 16)
    ts_ref[...] = (_fx16(ts[:, :d]) & 0xFFFF) | (_fx16(ts[:, d:]) << 16)


def _msg_body(pd_ref, ps_ref, ea_ref, w3_ref, b_ref, msg_ref):
    d = msg_ref.shape[1]
    e2 = jnp.dot(ea_ref[...], w3_ref[...],
                 preferred_element_type=jnp.float32) + b_ref[...]
    pd = pd_ref[...]
    ps = ps_ref[...]
    f = ((pd << 16) >> 16) + ((ps << 16) >> 16)
    s = (pd >> 16) + (ps >> 16)
    f = f.astype(jnp.float32) * (1.0 / _FXS) + e2[:, :d]
    s = s.astype(jnp.float32) * (1.0 / _FXS) + e2[:, d:]
    gate = 1.0 / (1.0 + jnp.exp(-f))
    sp = jnp.maximum(s, 0.0) + jnp.log1p(jnp.exp(-jnp.abs(s)))
    msg_ref[...] = gate * sp


def _final_body(p0_ref, p1_ref, x_ref, br_ref, bng_ref, bnb_ref, lng_ref,
                lnb_ref, wl_ref, bl_ref, g4_ref, b4_ref, out_ref):
    n = x_ref.shape[0]
    agg = ((p0_ref[0, :n] + p0_ref[1, :n])
           + (p1_ref[0, :n] + p1_ref[1, :n]))
    mu = jnp.mean(agg, axis=0, keepdims=True)
    var = jnp.mean((agg - mu) ** 2, axis=0, keepdims=True)
    aggn = (agg - mu) * lax.rsqrt(var + 1e-5) * bng_ref[...] + bnb_ref[...]
    h = x_ref[...] + aggn
    m = jnp.mean(h, axis=1, keepdims=True)
    v = jnp.mean((h - m) ** 2, axis=1, keepdims=True)
    h = (h - m) * lax.rsqrt(v + 1e-5) * lng_ref[...] + lnb_ref[...]
    h = jnp.maximum(h, 0.0) + jnp.log1p(jnp.exp(-jnp.abs(h)))
    bid = br_ref[...]
    g = out_ref.shape[0]
    gids = lax.broadcasted_iota(jnp.int32, (g, bid.shape[1]), 0)
    oh = (gids == bid).astype(jnp.float32)
    ssum = jnp.dot(oh, h, preferred_element_type=jnp.float32)
    cnt = jnp.sum(oh, axis=1, keepdims=True)
    smean = ssum / jnp.maximum(cnt, 1.0)
    pooled = jnp.concatenate([smean, ssum], axis=1)
    out = jnp.dot(pooled, wl_ref[...], preferred_element_type=jnp.float32) + bl_ref[...]
    m4 = jnp.mean(out, axis=1, keepdims=True)
    v4 = jnp.mean((out - m4) ** 2, axis=1, keepdims=True)
    out_ref[...] = (out - m4) * lax.rsqrt(v4 + 1e-5) * g4_ref[...] + b4_ref[...]


def _make_gather(n, e, z):
    epw = e // _NW
    ch = _CH
    nfull = epw // ch
    tail = epw - nfull * ch
    d = z // 2
    mesh = plsc.VectorSubcoreMesh(
        core_axis_name="c", subcore_axis_name="s",
        num_cores=_NC, num_subcores=_NS)

    @functools.partial(
        pl.kernel,
        out_type=[jax.ShapeDtypeStruct((e, d), jnp.int32),
                  jax.ShapeDtypeStruct((e, d), jnp.int32)],
        mesh=mesh,
        scratch_types=[
            pltpu.VMEM((epw,), jnp.int32),
            pltpu.VMEM((epw,), jnp.int32),
            pltpu.VMEM((ch, d), jnp.int32),
            pltpu.VMEM((ch, d), jnp.int32),
            pltpu.VMEM((ch, d), jnp.int32),
            pltpu.VMEM((ch, d), jnp.int32),
            pltpu.SemaphoreType.DMA,
            pltpu.SemaphoreType.DMA,
            pltpu.SemaphoreType.DMA,
            pltpu.SemaphoreType.DMA,
        ],
    )
    def gather_k(td_hbm, ts_hbm, dst_hbm, src_hbm, outd_hbm, outs_hbm,
                 idxd, idxs, bufd0, bufs0, bufd1, bufs1,
                 semd0, sems0, semd1, sems1):
        wid = lax.axis_index("s") * _NC + lax.axis_index("c")
        base = wid * epw
        bufd = (bufd0, bufd1)
        bufs = (bufs0, bufs1)
        semd = (semd0, semd1)
        sems = (sems0, sems1)
        pltpu.sync_copy(dst_hbm.at[pl.ds(base, epw)], idxd)
        pltpu.sync_copy(src_hbm.at[pl.ds(base, epw)], idxs)

        def issue(ci, b):
            pltpu.async_copy(
                td_hbm.at[idxd.at[pl.ds(ci * ch, ch)]], bufd[b], semd[b])
            pltpu.async_copy(
                ts_hbm.at[idxs.at[pl.ds(ci * ch, ch)]], bufs[b], sems[b])

        def wait(b):
            pltpu.make_async_copy(td_hbm.at[pl.ds(0, ch)], bufd[b], semd[b]).wait()
            pltpu.make_async_copy(ts_hbm.at[pl.ds(0, ch)], bufs[b], sems[b]).wait()

        def process(ci, b):
            pltpu.sync_copy(bufd[b], outd_hbm.at[pl.ds(base + ci * ch, ch)])
            pltpu.sync_copy(bufs[b], outs_hbm.at[pl.ds(base + ci * ch, ch)])

        issue(0, 0)
        issue(1, 1)

        def step(g2, _):
            for b in range(2):
                ci = g2 * 2 + b
                wait(b)
                process(ci, b)

                @pl.when(ci + 2 < nfull)
                def _():
                    issue(ci + 2, b)
            return 0
        lax.fori_loop(0, nfull // 2, step, 0)
        if nfull % 2:
            wait(0)
            process(nfull - 1, 0)

        if tail:
            b0 = base + nfull * ch
            pltpu.async_copy(
                td_hbm.at[idxd.at[pl.ds(nfull * ch - (ch - tail), ch)]],
                bufd[0], semd[0])
            pltpu.async_copy(
                ts_hbm.at[idxs.at[pl.ds(nfull * ch - (ch - tail), ch)]],
                bufs[0], sems[0])
            wait(0)
            pltpu.sync_copy(bufd[0].at[pl.ds(ch - tail, tail)],
                            outd_hbm.at[pl.ds(b0, tail)])
            pltpu.sync_copy(bufs[0].at[pl.ds(ch - tail, tail)],
                            outs_hbm.at[pl.ds(b0, tail)])

    return gather_k


def _make_scatter(n, e, d):
    epw = e // _NW
    nfull = epw // _CH
    tail = epw - nfull * _CH
    n_pad = -(-n // (_NS * 8)) * (_NS * 8)
    npt = n_pad // _NS
    zchunks = [_CH] * (npt // _CH)
    if npt % _CH:
        zchunks.append(npt % _CH)
    mesh = plsc.VectorSubcoreMesh(
        core_axis_name="c", subcore_axis_name="s",
        num_cores=_NC, num_subcores=_NS)

    scratch = [
        pltpu.VMEM((_CH,), jnp.int32),
        pltpu.VMEM((_CH,), jnp.int32),
        pltpu.VMEM((_CH, d), jnp.float32),
        pltpu.VMEM((_CH, d), jnp.float32),
        pltpu.VMEM_SHARED((n_pad, d), jnp.float32),
        pltpu.SemaphoreType.DMA,
        pltpu.SemaphoreType.DMA,
        pltpu.SemaphoreType.DMA,
        pltpu.SemaphoreType.DMA,
    ]
    if tail:
        scratch.append(pltpu.VMEM((tail,), jnp.int32))

    @functools.partial(
        pl.kernel,
        out_type=jax.ShapeDtypeStruct((_NC, n_pad, d), jnp.float32),
        mesh=mesh,
        scratch_types=scratch,
    )
    def scatter_k(msg_hbm, dst_hbm, out_hbm, idx0, idx1, buf0, buf1, shared,
                  semi0, semi1, sem0, sem1, *rest):
        idxt = rest[0] if tail else None
        cid = lax.axis_index("c")
        sid = lax.axis_index("s")
        wid = cid * _NS + sid
        base = wid * epw
        idx = (idx0, idx1)
        buf = (buf0, buf1)
        semi = (semi0, semi1)
        sem = (sem0, sem1)
        zero16 = jnp.zeros((_L,), jnp.float32)

        def zrow(r, _):
            for c in range(d // _L):
                buf0[r, pl.ds(c * _L, _L)] = zero16
            return 0
        lax.fori_loop(0, _CH, zrow, 0)
        off = 0
        for zc in zchunks:
            pltpu.sync_copy(buf0.at[pl.ds(0, zc)],
                            shared.at[pl.ds(sid * npt + off, zc)])
            off += zc
        plsc.subcore_barrier()

        def issue(ci, b):
            b0 = base + ci * _CH
            pltpu.async_copy(dst_hbm.at[pl.ds(b0, _CH)], idx[b], semi[b])
            pltpu.async_copy(msg_hbm.at[pl.ds(b0, _CH)], buf[b], sem[b])

        def wait(b):
            pltpu.make_async_copy(dst_hbm.at[pl.ds(0, _CH)],
                                  idx[b], semi[b]).wait()
            pltpu.make_async_copy(msg_hbm.at[pl.ds(0, _CH)],
                                  buf[b], sem[b]).wait()

        issue(0, 0)
        issue(1, 1)

        def step(g2, _):
            for b in range(2):
                ci = g2 * 2 + b
                wait(b)
                pltpu.sync_copy(buf[b], shared.at[idx[b]], add=True)

                @pl.when(ci + 2 < nfull)
                def _():
                    issue(ci + 2, b)
            return 0
        lax.fori_loop(0, nfull // 2, step, 0)
        if nfull % 2:
            wait(0)
            pltpu.sync_copy(buf[0], shared.at[idx[0]], add=True)

        if tail:
            b0 = base + nfull * _CH
            pltpu.sync_copy(dst_hbm.at[pl.ds(b0, tail)], idxt)
            pltpu.sync_copy(msg_hbm.at[pl.ds(b0, tail)], buf0.at[pl.ds(0, tail)])
            pltpu.sync_copy(buf0.at[pl.ds(0, tail)], shared.at[idxt], add=True)
        plsc.subcore_barrier()
        pltpu.sync_copy(shared.at[pl.ds(sid * npt, npt)],
                        out_hbm.at[cid, pl.ds(sid * npt, npt)])

    return scatter_k


def kernel(x, edge_index, edge_attr, batch, W_f, b_f, W_s, b_s,
           bn_gamma, bn_beta, ln_gamma, ln_beta, W_lin, b_lin,
           ln4_gamma, ln4_beta):
    n, d = x.shape
    e = edge_index.shape[1]
    de = edge_attr.shape[1]
    c = W_lin.shape[0]
    g = 64
    z = 2 * d

    src = edge_index[0]
    dst = edge_index[1]
    wd = jnp.concatenate([W_f[:, :d].T, W_s[:, :d].T], axis=1)
    wsrc = jnp.concatenate([W_f[:, d:2 * d].T, W_s[:, d:2 * d].T], axis=1)
    w3 = jnp.concatenate([W_f[:, 2 * d:].T, W_s[:, 2 * d:].T], axis=1)
    bias = jnp.concatenate([b_f, b_s])[None, :]

    bn = 2000
    t_dst, t_src = pl.pallas_call(
        _tables_body,
        grid=(n // bn,),
        in_specs=[
            pl.BlockSpec((bn, d), lambda i: (i, 0)),
            pl.BlockSpec((d, z), lambda i: (0, 0)),
            pl.BlockSpec((d, z), lambda i: (0, 0)),
        ],
        out_specs=[
            pl.BlockSpec((bn, d), lambda i: (i, 0)),
            pl.BlockSpec((bn, d), lambda i: (i, 0)),
        ],
        out_shape=[
            jax.ShapeDtypeStruct((n, d), jnp.int32),
            jax.ShapeDtypeStruct((n, d), jnp.int32),
        ],
    )(x, wd, wsrc)

    e2 = e // 2
    be = 2000
    gather_fn = _make_gather(n, e2, z)
    scatter_fn = _make_scatter(n, e2, d)

    def msg_fn(pre_d, pre_s, ea_sl):
        return pl.pallas_call(
            _msg_body,
            grid=(e2 // be,),
            in_specs=[
                pl.BlockSpec((be, d), lambda i: (i, 0)),
                pl.BlockSpec((be, d), lambda i: (i, 0)),
                pl.BlockSpec((be, de), lambda i: (i, 0)),
                pl.BlockSpec((de, z), lambda i: (0, 0)),
                pl.BlockSpec((1, z), lambda i: (0, 0)),
            ],
            out_specs=pl.BlockSpec((be, d), lambda i: (i, 0)),
            out_shape=jax.ShapeDtypeStruct((e2, d), jnp.float32),
        )(pre_d, pre_s, ea_sl, w3, bias)

    s0, s1 = slice(0, e2), slice(e2, e)
    pre_d0, pre_s0 = gather_fn(t_dst, t_src, dst[s0], src[s0])
    tie_g1 = pre_d0[0, 0] * 0
    pre_d1, pre_s1 = gather_fn(t_dst, t_src, dst[s1] + tie_g1, src[s1])
    msg0 = msg_fn(pre_d0, pre_s0, edge_attr[s0])
    tie_s0 = pre_d1[0, 0] * 0
    part0 = scatter_fn(msg0, dst[s0] + tie_s0)
    msg1 = msg_fn(pre_d1, pre_s1, edge_attr[s1])
    tie_s1 = lax.convert_element_type(part0[0, 0, 0] * 0, jnp.int32)
    part1 = scatter_fn(msg1, dst[s1] + tie_s1)
    partials = [part0, part1]

    out = pl.pallas_call(
        _final_body,
        out_shape=jax.ShapeDtypeStruct((g, c), jnp.float32),
    )(partials[0], partials[1], x, batch[None, :].astype(jnp.int32),
      bn_gamma[None, :], bn_beta[None, :], ln_gamma[None, :], ln_beta[None, :],
      W_lin.T, b_lin[None, :], ln4_gamma[None, :], ln4_beta[None, :])
    return out


# speedup vs baseline: 1.2029x; 1.2029x over previous
import functools

import jax
import jax.numpy as jnp
from jax import lax
from jax.experimental import pallas as pl
from jax.experimental.pallas import tpu as pltpu
from jax.experimental.pallas import tpu_sc as plsc

_NC = 2
_NS = 16
_NW = _NC * _NS
_CH = 128
_L = 16


_FXS = 2048.0
_MSIGN = -2147450880
_MVAL = 0x7FFF7FFF


def _fx16(v):
    return jnp.clip(jnp.round(v * _FXS), -32768.0, 32767.0).astype(jnp.int32)


def _tables_body(x_ref, wd_ref, ws_ref, td_ref, ts_ref):
    xb = x_ref[...]
    d = xb.shape[1]
    td = jnp.dot(xb, wd_ref[...], preferred_element_type=jnp.float32)
    ts = jnp.dot(xb, ws_ref[...], preferred_element_type=jnp.float32)
    td_ref[...] = (_fx16(td[:, :d]) & 0xFFFF) | (_fx16(td[:, d:]) << 16)
    ts_ref[...] = (_fx16(ts[:, :d]) & 0xFFFF) | (_fx16(ts[:, d:]) << 16)


def _msg_body(p_ref, ea_ref, w3_ref, b_ref, msg_ref):
    d = msg_ref.shape[1]
    e2 = jnp.dot(ea_ref[...], w3_ref[...],
                 preferred_element_type=jnp.float32) + b_ref[...]
    p = p_ref[...]
    f = ((p << 16) >> 16).astype(jnp.float32) * (1.0 / _FXS) + e2[:, :d]
    s = (p >> 16).astype(jnp.float32) * (1.0 / _FXS) + e2[:, d:]
    gate = 1.0 / (1.0 + jnp.exp(-f))
    sp = jnp.maximum(s, 0.0) + jnp.log1p(jnp.exp(-jnp.abs(s)))
    msg_ref[...] = gate * sp


def _final_body(p_ref, x_ref, br_ref, bng_ref, bnb_ref, lng_ref,
                lnb_ref, wl_ref, bl_ref, g4_ref, b4_ref, out_ref):
    n = x_ref.shape[0]
    agg = p_ref[0, :n] + p_ref[1, :n]
    mu = jnp.mean(agg, axis=0, keepdims=True)
    var = jnp.mean((agg - mu) ** 2, axis=0, keepdims=True)
    aggn = (agg - mu) * lax.rsqrt(var + 1e-5) * bng_ref[...] + bnb_ref[...]
    h = x_ref[...] + aggn
    m = jnp.mean(h, axis=1, keepdims=True)
    v = jnp.mean((h - m) ** 2, axis=1, keepdims=True)
    h = (h - m) * lax.rsqrt(v + 1e-5) * lng_ref[...] + lnb_ref[...]
    h = jnp.maximum(h, 0.0) + jnp.log1p(jnp.exp(-jnp.abs(h)))
    bid = br_ref[...]
    g = out_ref.shape[0]
    gids = lax.broadcasted_iota(jnp.int32, (g, bid.shape[1]), 0)
    oh = (gids == bid).astype(jnp.float32)
    ssum = jnp.dot(oh, h, preferred_element_type=jnp.float32)
    cnt = jnp.sum(oh, axis=1, keepdims=True)
    smean = ssum / jnp.maximum(cnt, 1.0)
    pooled = jnp.concatenate([smean, ssum], axis=1)
    out = jnp.dot(pooled, wl_ref[...], preferred_element_type=jnp.float32) + bl_ref[...]
    m4 = jnp.mean(out, axis=1, keepdims=True)
    v4 = jnp.mean((out - m4) ** 2, axis=1, keepdims=True)
    out_ref[...] = (out - m4) * lax.rsqrt(v4 + 1e-5) * g4_ref[...] + b4_ref[...]


def _make_gather(n, e, z):
    epw = e // _NW
    ch = _CH
    nfull = epw // ch
    tail = epw - nfull * ch
    d = z // 2
    mesh = plsc.VectorSubcoreMesh(
        core_axis_name="c", subcore_axis_name="s",
        num_cores=_NC, num_subcores=_NS)

    @functools.partial(
        pl.kernel,
        out_type=jax.ShapeDtypeStruct((e, d), jnp.int32),
        mesh=mesh,
        scratch_types=[
            pltpu.VMEM((epw,), jnp.int32),
            pltpu.VMEM((epw,), jnp.int32),
            pltpu.VMEM((ch, d), jnp.int32),
            pltpu.VMEM((ch, d), jnp.int32),
            pltpu.VMEM((ch, d), jnp.int32),
            pltpu.VMEM((ch, d), jnp.int32),
            pltpu.SemaphoreType.DMA,
            pltpu.SemaphoreType.DMA,
            pltpu.SemaphoreType.DMA,
            pltpu.SemaphoreType.DMA,
        ],
    )
    def gather_k(td_hbm, ts_hbm, dst_hbm, src_hbm, out_hbm,
                 idxd, idxs, bufd0, bufs0, bufd1, bufs1,
                 semd0, sems0, semd1, sems1):
        wid = lax.axis_index("s") * _NC + lax.axis_index("c")
        base = wid * epw
        bufd = (bufd0, bufd1)
        bufs = (bufs0, bufs1)
        semd = (semd0, semd1)
        sems = (sems0, sems1)
        pltpu.sync_copy(dst_hbm.at[pl.ds(base, epw)], idxd)
        pltpu.sync_copy(src_hbm.at[pl.ds(base, epw)], idxs)

        def issue(ci, b):
            pltpu.async_copy(
                td_hbm.at[idxd.at[pl.ds(ci * ch, ch)]], bufd[b], semd[b])
            pltpu.async_copy(
                ts_hbm.at[idxs.at[pl.ds(ci * ch, ch)]], bufs[b], sems[b])

        def wait(b):
            pltpu.make_async_copy(td_hbm.at[pl.ds(0, ch)], bufd[b], semd[b]).wait()
            pltpu.make_async_copy(ts_hbm.at[pl.ds(0, ch)], bufs[b], sems[b]).wait()

        def add_rows(b, lo, hi):
            def add_row(r, _):
                for c in range(d // _L):
                    sl = pl.ds(c * _L, _L)
                    va = bufd[b][r, sl]
                    vb = bufs[b][r, sl]
                    bufd[b][r, sl] = (((va & _MVAL) + (vb & _MVAL))
                                      ^ ((va ^ vb) & _MSIGN))
                return 0
            lax.fori_loop(lo, hi, add_row, 0)

        def process(ci, b):
            add_rows(b, 0, ch)
            pltpu.sync_copy(bufd[b], out_hbm.at[pl.ds(base + ci * ch, ch)])

        issue(0, 0)
        issue(1, 1)

        def step(g2, _):
            for b in range(2):
                ci = g2 * 2 + b
                wait(b)
                process(ci, b)

                @pl.when(ci + 2 < nfull)
                def _():
                    issue(ci + 2, b)
            return 0
        lax.fori_loop(0, nfull // 2, step, 0)
        if nfull % 2:
            wait(0)
            process(nfull - 1, 0)

        if tail:
            b0 = base + nfull * ch
            pltpu.async_copy(
                td_hbm.at[idxd.at[pl.ds(nfull * ch - (ch - tail), ch)]],
                bufd[0], semd[0])
            pltpu.async_copy(
                ts_hbm.at[idxs.at[pl.ds(nfull * ch - (ch - tail), ch)]],
                bufs[0], sems[0])
            wait(0)
            add_rows(0, ch - tail, ch)
            pltpu.sync_copy(bufd[0].at[pl.ds(ch - tail, tail)],
                            out_hbm.at[pl.ds(b0, tail)])

    return gather_k


def _make_scatter(n, e, d):
    epw = e // _NW
    nfull = epw // _CH
    tail = epw - nfull * _CH
    n_pad = -(-n // (_NS * 8)) * (_NS * 8)
    npt = n_pad // _NS
    zchunks = [_CH] * (npt // _CH)
    if npt % _CH:
        zchunks.append(npt % _CH)
    mesh = plsc.VectorSubcoreMesh(
        core_axis_name="c", subcore_axis_name="s",
        num_cores=_NC, num_subcores=_NS)

    scratch = [
        pltpu.VMEM((_CH,), jnp.int32),
        pltpu.VMEM((_CH,), jnp.int32),
        pltpu.VMEM((_CH, d), jnp.float32),
        pltpu.VMEM((_CH, d), jnp.float32),
        pltpu.VMEM_SHARED((n_pad, d), jnp.float32),
        pltpu.SemaphoreType.DMA,
        pltpu.SemaphoreType.DMA,
        pltpu.SemaphoreType.DMA,
        pltpu.SemaphoreType.DMA,
    ]
    if tail:
        scratch.append(pltpu.VMEM((tail,), jnp.int32))

    @functools.partial(
        pl.kernel,
        out_type=jax.ShapeDtypeStruct((_NC, n_pad, d), jnp.float32),
        mesh=mesh,
        scratch_types=scratch,
    )
    def scatter_k(msg_hbm, dst_hbm, out_hbm, idx0, idx1, buf0, buf1, shared,
                  semi0, semi1, sem0, sem1, *rest):
        idxt = rest[0] if tail else None
        cid = lax.axis_index("c")
        sid = lax.axis_index("s")
        wid = cid * _NS + sid
        base = wid * epw
        idx = (idx0, idx1)
        buf = (buf0, buf1)
        semi = (semi0, semi1)
        sem = (sem0, sem1)
        zero16 = jnp.zeros((_L,), jnp.float32)

        def zrow(r, _):
            for c in range(d // _L):
                buf0[r, pl.ds(c * _L, _L)] = zero16
            return 0
        lax.fori_loop(0, _CH, zrow, 0)
        off = 0
        for zc in zchunks:
            pltpu.sync_copy(buf0.at[pl.ds(0, zc)],
                            shared.at[pl.ds(sid * npt + off, zc)])
            off += zc
        plsc.subcore_barrier()

        def issue(ci, b):
            b0 = base + ci * _CH
            pltpu.async_copy(dst_hbm.at[pl.ds(b0, _CH)], idx[b], semi[b])
            pltpu.async_copy(msg_hbm.at[pl.ds(b0, _CH)], buf[b], sem[b])

        def wait(b):
            pltpu.make_async_copy(dst_hbm.at[pl.ds(0, _CH)],
                                  idx[b], semi[b]).wait()
            pltpu.make_async_copy(msg_hbm.at[pl.ds(0, _CH)],
                                  buf[b], sem[b]).wait()

        issue(0, 0)
        issue(1, 1)

        def step(g2, _):
            for b in range(2):
                ci = g2 * 2 + b
                wait(b)
                pltpu.sync_copy(buf[b], shared.at[idx[b]], add=True)

                @pl.when(ci + 2 < nfull)
                def _():
                    issue(ci + 2, b)
            return 0
        lax.fori_loop(0, nfull // 2, step, 0)
        if nfull % 2:
            wait(0)
            pltpu.sync_copy(buf[0], shared.at[idx[0]], add=True)

        if tail:
            b0 = base + nfull * _CH
            pltpu.sync_copy(dst_hbm.at[pl.ds(b0, tail)], idxt)
            pltpu.sync_copy(msg_hbm.at[pl.ds(b0, tail)], buf0.at[pl.ds(0, tail)])
            pltpu.sync_copy(buf0.at[pl.ds(0, tail)], shared.at[idxt], add=True)
        plsc.subcore_barrier()
        pltpu.sync_copy(shared.at[pl.ds(sid * npt, npt)],
                        out_hbm.at[cid, pl.ds(sid * npt, npt)])

    return scatter_k


def kernel(x, edge_index, edge_attr, batch, W_f, b_f, W_s, b_s,
           bn_gamma, bn_beta, ln_gamma, ln_beta, W_lin, b_lin,
           ln4_gamma, ln4_beta):
    n, d = x.shape
    e = edge_index.shape[1]
    de = edge_attr.shape[1]
    c = W_lin.shape[0]
    g = 64
    z = 2 * d

    src = edge_index[0]
    dst = edge_index[1]
    wd = jnp.concatenate([W_f[:, :d].T, W_s[:, :d].T], axis=1)
    wsrc = jnp.concatenate([W_f[:, d:2 * d].T, W_s[:, d:2 * d].T], axis=1)
    w3 = jnp.concatenate([W_f[:, 2 * d:].T, W_s[:, 2 * d:].T], axis=1)
    bias = jnp.concatenate([b_f, b_s])[None, :]

    bn = 2000
    t_dst, t_src = pl.pallas_call(
        _tables_body,
        grid=(n // bn,),
        in_specs=[
            pl.BlockSpec((bn, d), lambda i: (i, 0)),
            pl.BlockSpec((d, z), lambda i: (0, 0)),
            pl.BlockSpec((d, z), lambda i: (0, 0)),
        ],
        out_specs=[
            pl.BlockSpec((bn, d), lambda i: (i, 0)),
            pl.BlockSpec((bn, d), lambda i: (i, 0)),
        ],
        out_shape=[
            jax.ShapeDtypeStruct((n, d), jnp.int32),
            jax.ShapeDtypeStruct((n, d), jnp.int32),
        ],
    )(x, wd, wsrc)

    pre = _make_gather(n, e, z)(t_dst, t_src, dst, src)

    be = 2000
    msg = pl.pallas_call(
        _msg_body,
        grid=(e // be,),
        in_specs=[
            pl.BlockSpec((be, d), lambda i: (i, 0)),
            pl.BlockSpec((be, de), lambda i: (i, 0)),
            pl.BlockSpec((de, z), lambda i: (0, 0)),
            pl.BlockSpec((1, z), lambda i: (0, 0)),
        ],
        out_specs=pl.BlockSpec((be, d), lambda i: (i, 0)),
        out_shape=jax.ShapeDtypeStruct((e, d), jnp.float32),
    )(pre, edge_attr, w3, bias)

    partials = _make_scatter(n, e, d)(msg, dst)

    out = pl.pallas_call(
        _final_body,
        out_shape=jax.ShapeDtypeStruct((g, c), jnp.float32),
    )(partials, x, batch[None, :].astype(jnp.int32),
      bn_gamma[None, :], bn_beta[None, :], ln_gamma[None, :], ln_beta[None, :],
      W_lin.T, b_lin[None, :], ln4_gamma[None, :], ln4_beta[None, :])
    return out
